# Initial kernel scaffold; baseline (speedup 1.0000x reference)
import jax
import jax.numpy as jnp
from jax.experimental import pallas as pl


def kernel(x):
    raise NotImplementedError("write your pallas kernel here")


# speedup vs baseline: 2.4773x; 2.4773x over previous
import functools

import jax
import jax.numpy as jnp
from jax import lax
from jax.experimental import pallas as pl
from jax.experimental.pallas import tpu as pltpu
from jax.experimental.pallas import tpu_sc as plsc

R = 128
N = 32768
K = 32
L = 16
NC, NS = 2, 16
NW = NC * NS
RPW = R // NW
NCHUNK = N // L
UNROLL = 8
CAP = 2048

_NEG_INF = float("-inf")
_BIG_IDX = 2**30


def _topk_body(x_hbm, val_hbm, idx_hbm, row_v, cval_v, cidx_v, oval_v, oidx_v):
    wid = lax.axis_index("s") * NC + lax.axis_index("c")

    def do_row(r, _):
        row = wid * RPW + r
        pltpu.sync_copy(x_hbm.at[row], row_v)

        def p1_body(i, accs):
            base = i * (UNROLL * L)
            return tuple(
                jnp.maximum(accs[j], row_v[pl.ds(base + j * L, L)])
                for j in range(UNROLL)
            )

        init = tuple(jnp.full((L,), _NEG_INF, dtype=jnp.float32) for _ in range(UNROLL))
        accs = lax.fori_loop(0, NCHUNK // UNROLL, p1_body, init)
        a = jnp.maximum(jnp.maximum(accs[0], accs[1]),
                        jnp.maximum(accs[2], accs[3]))
        b = jnp.maximum(jnp.maximum(accs[4], accs[5]),
                        jnp.maximum(accs[6], accs[7]))
        t = jnp.minimum(jnp.min(a), jnp.min(b))

        lane_iota = lax.iota(jnp.int32, L)

        def p2_body(i, off):
            v = row_v[pl.ds(i * L, L)]
            mask = v >= t
            idx = i * L + lane_iota
            plsc.store_compressed(cval_v.at[pl.ds(off, L)], v, mask=mask)
            plsc.store_compressed(cidx_v.at[pl.ds(off, L)], idx, mask=mask)
            cnt = jnp.sum(mask.astype(jnp.int32))
            return jnp.minimum(off + cnt, CAP)

        n = lax.fori_loop(0, NCHUNK, p2_body, jnp.int32(0))
        cval_v[pl.ds(n, L)] = jnp.full((L,), _NEG_INF, dtype=jnp.float32)
        nv = (n + L - 1) // L

        def sel_step(k, carry):
            pv, pi, ov0, ov1, oi0, oi1 = carry

            def scan_vregs(j, best):
                bv, bi = best
                v = cval_v[pl.ds(j * L, L)]
                ii = cidx_v[pl.ds(j * L, L)]
                elig = (v < pv) | ((v == pv) & (ii > pi))
                v2 = jnp.where(elig, v, _NEG_INF)
                take = (v2 > bv) | ((v2 == bv) & (ii < bi))
                return (jnp.where(take, v2, bv), jnp.where(take, ii, bi))

            binit = (jnp.full((L,), _NEG_INF, dtype=jnp.float32),
                     jnp.full((L,), _BIG_IDX, dtype=jnp.int32))
            bv, bi = lax.fori_loop(0, nv, scan_vregs, binit)
            best_val = jnp.max(bv)
            best_idx = jnp.min(jnp.where(bv == best_val, bi, _BIG_IDX))
            slot0 = (k < L) & (lane_iota == k)
            slot1 = (k >= L) & (lane_iota == k - L)
            ov0 = jnp.where(slot0, best_val, ov0)
            ov1 = jnp.where(slot1, best_val, ov1)
            oi0 = jnp.where(slot0, best_idx, oi0)
            oi1 = jnp.where(slot1, best_idx, oi1)
            return (best_val, best_idx, ov0, ov1, oi0, oi1)

        zf = jnp.zeros((L,), dtype=jnp.float32)
        zi = jnp.zeros((L,), dtype=jnp.int32)
        _, _, ov0, ov1, oi0, oi1 = lax.fori_loop(
            0, K, sel_step,
            (jnp.float32(jnp.inf), jnp.int32(-1), zf, zf, zi, zi))
        oval_v[pl.ds(0, L)] = ov0
        oval_v[pl.ds(L, L)] = ov1
        oidx_v[pl.ds(0, L)] = oi0
        oidx_v[pl.ds(L, L)] = oi1

        pltpu.sync_copy(oval_v, val_hbm.at[row])
        pltpu.sync_copy(oidx_v, idx_hbm.at[row])
        return 0

    lax.fori_loop(0, RPW, do_row, 0)


@jax.jit
def kernel(x):
    mesh = plsc.VectorSubcoreMesh(
        core_axis_name="c", subcore_axis_name="s",
        num_cores=NC, num_subcores=NS)
    f = pl.kernel(
        _topk_body,
        out_type=(
            jax.ShapeDtypeStruct((R, K), jnp.float32),
            jax.ShapeDtypeStruct((R, K), jnp.int32),
        ),
        compiler_params=pltpu.CompilerParams(needs_layout_passes=False),
        mesh=mesh,
        scratch_types=[
            pltpu.VMEM((N,), jnp.float32),
            pltpu.VMEM((CAP + L,), jnp.float32),
            pltpu.VMEM((CAP + L,), jnp.int32),
            pltpu.VMEM((K,), jnp.float32),
            pltpu.VMEM((K,), jnp.int32),
        ],
    )
    return f(x)
